# Initial kernel scaffold; baseline (speedup 1.0000x reference)
import jax
import jax.numpy as jnp
from jax.experimental import pallas as pl


def kernel(eb_we, eb_ws, eb_wr, eb_wu, eb_b, nb_wagg, nb_wx, nb_wu, nb_b, gb_we, gb_wn, gb_wu, gb_b, dec1_we, dec1_ws, dec1_wr, dec1_wu, dec1_b, dec2_w, dec2_b, x, edge_index, edge_attr, u):
    raise NotImplementedError("write your pallas kernel here")


# speedup vs baseline: 10.7962x; 10.7962x over previous
import functools

import jax
import jax.numpy as jnp
from jax.experimental import pallas as pl
from jax.experimental.pallas import tpu as pltpu

_CompilerParams = getattr(pltpu, "CompilerParams", None) or getattr(pltpu, "TPUCompilerParams")

_VMEM_LIMIT = 64 * 1024 * 1024


def _edge_encode_kernel(e_ref, xs_ref, xr_ref, u_ref,
                        we_ref, ws_ref, wr_ref, wu_ref, b_ref,
                        eh_ref, esum_ref):
    y = (jnp.dot(e_ref[...], we_ref[...], preferred_element_type=jnp.float32)
         + jnp.dot(xs_ref[...], ws_ref[...], preferred_element_type=jnp.float32)
         + jnp.dot(xr_ref[...], wr_ref[...], preferred_element_type=jnp.float32)
         + jnp.dot(u_ref[...], wu_ref[...], preferred_element_type=jnp.float32)
         + b_ref[...])
    e_h = jnp.maximum(y, 0.0)
    eh_ref[...] = e_h

    @pl.when(pl.program_id(1) == 0)
    def _():
        esum_ref[...] = jnp.zeros_like(esum_ref)

    esum_ref[...] += jnp.sum(e_h, axis=0, keepdims=True)


def _node_base_kernel(x_ref, u_ref, wx_ref, wu_ref, b_ref, nsum_ref):
    c = jnp.dot(u_ref[...], wu_ref[...], preferred_element_type=jnp.float32) + b_ref[...]
    nb = jnp.maximum(
        jnp.dot(x_ref[...], wx_ref[...], preferred_element_type=jnp.float32) + c, 0.0)

    @pl.when(pl.program_id(1) == 0)
    def _():
        nsum_ref[...] = jnp.zeros_like(nsum_ref)

    nsum_ref[...] += jnp.sum(nb, axis=0, keepdims=True)


def _edge_node_kernel(rc_ref, sc_ref, rrow_ref, eh_ref, xr_ref, xs_ref, u_ref,
                      wagg_ref, wx_ref, wu_ref, b_ref,
                      nr_ref, ns_ref, corr_ref):
    c = jnp.dot(u_ref[...], wu_ref[...], preferred_element_type=jnp.float32) + b_ref[...]
    rrow = rrow_ref[...]
    eh16 = eh_ref[...].astype(jnp.bfloat16)

    m_r = rc_ref[...] == rrow
    aggr = jnp.dot(m_r.astype(jnp.bfloat16), eh16,
                   preferred_element_type=jnp.float32)
    mult = jnp.sum(m_r.astype(jnp.float32), axis=1, keepdims=True)

    m_s = sc_ref[...] == rrow
    aggs = jnp.dot(m_s.astype(jnp.bfloat16), eh16,
                   preferred_element_type=jnp.float32)

    base_r = jnp.dot(xr_ref[...], wx_ref[...], preferred_element_type=jnp.float32) + c
    base_s = jnp.dot(xs_ref[...], wx_ref[...], preferred_element_type=jnp.float32) + c
    nr = jnp.maximum(
        base_r + jnp.dot(aggr, wagg_ref[...], preferred_element_type=jnp.float32), 0.0)
    ns = jnp.maximum(
        base_s + jnp.dot(aggs, wagg_ref[...], preferred_element_type=jnp.float32), 0.0)
    nr_ref[...] = nr
    ns_ref[...] = ns

    delta = (nr - jnp.maximum(base_r, 0.0)) / mult

    @pl.when(pl.program_id(1) == 0)
    def _():
        corr_ref[...] = jnp.zeros_like(corr_ref)

    corr_ref[...] += jnp.sum(delta, axis=0, keepdims=True)


def _global_kernel(esum_ref, nsum_ref, corr_ref, u_ref,
                   we_ref, wn_ref, wu_ref, b_ref, uh_ref, *, inv_e, inv_n):
    e_mean = jnp.sum(esum_ref[...], axis=0) * inv_e
    n_sum = (jnp.sum(nsum_ref[...], axis=0)
             + jnp.sum(corr_ref[...], axis=0))
    n_mean = n_sum * inv_n
    y = (jnp.dot(e_mean, we_ref[...], preferred_element_type=jnp.float32)
         + jnp.dot(n_mean, wn_ref[...], preferred_element_type=jnp.float32)
         + jnp.dot(u_ref[...], wu_ref[...], preferred_element_type=jnp.float32)
         + b_ref[...])
    uh_ref[...] = jnp.maximum(y, 0.0)


def _decode_kernel(eh_ref, ns_ref, nr_ref, uh_ref, xrs_ref,
                   w1e_ref, w1s_ref, w1r_ref, w1u_ref, b1_ref,
                   w2_ref, b2_ref, out_ref, *, out_dim):
    h = (jnp.dot(eh_ref[...], w1e_ref[...], preferred_element_type=jnp.float32)
         + jnp.dot(ns_ref[...], w1s_ref[...], preferred_element_type=jnp.float32)
         + jnp.dot(nr_ref[...], w1r_ref[...], preferred_element_type=jnp.float32)
         + jnp.dot(uh_ref[...], w1u_ref[...], preferred_element_type=jnp.float32)
         + b1_ref[...])
    h = jnp.maximum(h, 0.0)
    o = jnp.dot(h, w2_ref[...], preferred_element_type=jnp.float32) + b2_ref[...]
    o1 = o[:, :out_dim]
    o2 = o[:, out_dim:]
    xrs = xrs_ref[...]
    out_ref[...] = o1 * (xrs[:, 0:1] - o2 * xrs[:, 1:2])


def _split(dim, tile):
    t = min(tile, dim)
    if dim % t != 0 or t % 8 != 0:
        t = dim
    steps = dim // t
    cores = 2 if steps % 2 == 0 else 1
    return t, cores, steps // cores


def kernel(eb_we, eb_ws, eb_wr, eb_wu, eb_b,
           nb_wagg, nb_wx, nb_wu, nb_b,
           gb_we, gb_wn, gb_wu, gb_b,
           dec1_we, dec1_ws, dec1_wr, dec1_wu, dec1_b,
           dec2_w, dec2_b,
           x, edge_index, edge_attr, u):
    sind, rind = edge_index[0], edge_index[1]
    N, Fn = x.shape
    E, Fe = edge_attr.shape
    GH = u.shape[1]
    EH2 = eb_b.shape[1]
    NH2 = nb_b.shape[1]
    GH2 = gb_b.shape[1]
    OUT2 = dec2_b.shape[1]
    OUT = OUT2 // 2

    xs = x[sind]
    xr = x[rind]
    rrow = rind.astype(jnp.int32).reshape(1, E)
    rcol = rind.astype(jnp.int32).reshape(E, 1)
    scol = sind.astype(jnp.int32).reshape(E, 1)
    xrs = jnp.concatenate([x[rind, 2:3], x[sind, 2:3]], axis=1)

    te1, c1, j1 = _split(E, 2048)
    e_h, e_sum = pl.pallas_call(
        _edge_encode_kernel,
        grid=(c1, j1),
        out_shape=(jax.ShapeDtypeStruct((E, EH2), jnp.float32),
                   jax.ShapeDtypeStruct((c1, 1, EH2), jnp.float32)),
        in_specs=[
            pl.BlockSpec((te1, Fe), lambda i, j, J=j1: (i * J + j, 0)),
            pl.BlockSpec((te1, Fn), lambda i, j, J=j1: (i * J + j, 0)),
            pl.BlockSpec((te1, Fn), lambda i, j, J=j1: (i * J + j, 0)),
            pl.BlockSpec((1, GH), lambda i, j: (0, 0)),
            pl.BlockSpec((Fe, EH2), lambda i, j: (0, 0)),
            pl.BlockSpec((Fn, EH2), lambda i, j: (0, 0)),
            pl.BlockSpec((Fn, EH2), lambda i, j: (0, 0)),
            pl.BlockSpec((GH, EH2), lambda i, j: (0, 0)),
            pl.BlockSpec((1, EH2), lambda i, j: (0, 0)),
        ],
        out_specs=(pl.BlockSpec((te1, EH2), lambda i, j, J=j1: (i * J + j, 0)),
                   pl.BlockSpec((1, 1, EH2), lambda i, j: (i, 0, 0))),
        compiler_params=_CompilerParams(
            dimension_semantics=("parallel", "arbitrary"),
            vmem_limit_bytes=_VMEM_LIMIT),
    )(edge_attr, xs, xr, u, eb_we, eb_ws, eb_wr, eb_wu, eb_b)

    tn, c2, j2 = _split(N, 8192)
    nsum_base = pl.pallas_call(
        _node_base_kernel,
        grid=(c2, j2),
        out_shape=jax.ShapeDtypeStruct((c2, 1, NH2), jnp.float32),
        in_specs=[
            pl.BlockSpec((tn, Fn), lambda i, j, J=j2: (i * J + j, 0)),
            pl.BlockSpec((1, GH), lambda i, j: (0, 0)),
            pl.BlockSpec((Fn, NH2), lambda i, j: (0, 0)),
            pl.BlockSpec((GH, NH2), lambda i, j: (0, 0)),
            pl.BlockSpec((1, NH2), lambda i, j: (0, 0)),
        ],
        out_specs=pl.BlockSpec((1, 1, NH2), lambda i, j: (i, 0, 0)),
        compiler_params=_CompilerParams(
            dimension_semantics=("parallel", "arbitrary"),
            vmem_limit_bytes=_VMEM_LIMIT),
    )(x, u, nb_wx, nb_wu, nb_b)

    te3, c3, j3 = _split(E, 256)
    n_r, n_s, corr = pl.pallas_call(
        _edge_node_kernel,
        grid=(c3, j3),
        out_shape=(jax.ShapeDtypeStruct((E, NH2), jnp.float32),
                   jax.ShapeDtypeStruct((E, NH2), jnp.float32),
                   jax.ShapeDtypeStruct((c3, 1, NH2), jnp.float32)),
        in_specs=[
            pl.BlockSpec((te3, 1), lambda i, j, J=j3: (i * J + j, 0)),
            pl.BlockSpec((te3, 1), lambda i, j, J=j3: (i * J + j, 0)),
            pl.BlockSpec((1, E), lambda i, j: (0, 0)),
            pl.BlockSpec((E, EH2), lambda i, j: (0, 0)),
            pl.BlockSpec((te3, Fn), lambda i, j, J=j3: (i * J + j, 0)),
            pl.BlockSpec((te3, Fn), lambda i, j, J=j3: (i * J + j, 0)),
            pl.BlockSpec((1, GH), lambda i, j: (0, 0)),
            pl.BlockSpec((EH2, NH2), lambda i, j: (0, 0)),
            pl.BlockSpec((Fn, NH2), lambda i, j: (0, 0)),
            pl.BlockSpec((GH, NH2), lambda i, j: (0, 0)),
            pl.BlockSpec((1, NH2), lambda i, j: (0, 0)),
        ],
        out_specs=(pl.BlockSpec((te3, NH2), lambda i, j, J=j3: (i * J + j, 0)),
                   pl.BlockSpec((te3, NH2), lambda i, j, J=j3: (i * J + j, 0)),
                   pl.BlockSpec((1, 1, NH2), lambda i, j: (i, 0, 0))),
        compiler_params=_CompilerParams(
            dimension_semantics=("parallel", "arbitrary"),
            vmem_limit_bytes=_VMEM_LIMIT),
    )(rcol, scol, rrow, e_h, xr, xs, u, nb_wagg, nb_wx, nb_wu, nb_b)

    u_h = pl.pallas_call(
        functools.partial(_global_kernel, inv_e=1.0 / E, inv_n=1.0 / N),
        out_shape=jax.ShapeDtypeStruct((1, GH2), jnp.float32),
        in_specs=[
            pl.BlockSpec((c1, 1, EH2), lambda: (0, 0, 0)),
            pl.BlockSpec((c2, 1, NH2), lambda: (0, 0, 0)),
            pl.BlockSpec((c3, 1, NH2), lambda: (0, 0, 0)),
            pl.BlockSpec((1, GH), lambda: (0, 0)),
            pl.BlockSpec((EH2, GH2), lambda: (0, 0)),
            pl.BlockSpec((NH2, GH2), lambda: (0, 0)),
            pl.BlockSpec((GH, GH2), lambda: (0, 0)),
            pl.BlockSpec((1, GH2), lambda: (0, 0)),
        ],
        out_specs=pl.BlockSpec((1, GH2), lambda: (0, 0)),
    )(e_sum, nsum_base, corr, u, gb_we, gb_wn, gb_wu, gb_b)

    te5, _, _ = _split(E, 2048)
    out = pl.pallas_call(
        functools.partial(_decode_kernel, out_dim=OUT),
        grid=(E // te5,),
        out_shape=jax.ShapeDtypeStruct((E, OUT), jnp.float32),
        in_specs=[
            pl.BlockSpec((te5, EH2), lambda i: (i, 0)),
            pl.BlockSpec((te5, NH2), lambda i: (i, 0)),
            pl.BlockSpec((te5, NH2), lambda i: (i, 0)),
            pl.BlockSpec((1, GH2), lambda i: (0, 0)),
            pl.BlockSpec((te5, 2), lambda i: (i, 0)),
            pl.BlockSpec((EH2, EH2), lambda i: (0, 0)),
            pl.BlockSpec((NH2, EH2), lambda i: (0, 0)),
            pl.BlockSpec((NH2, EH2), lambda i: (0, 0)),
            pl.BlockSpec((GH2, EH2), lambda i: (0, 0)),
            pl.BlockSpec((1, EH2), lambda i: (0, 0)),
            pl.BlockSpec((EH2, OUT2), lambda i: (0, 0)),
            pl.BlockSpec((1, OUT2), lambda i: (0, 0)),
        ],
        out_specs=pl.BlockSpec((te5, OUT), lambda i: (i, 0)),
        compiler_params=_CompilerParams(
            dimension_semantics=("parallel",),
            vmem_limit_bytes=_VMEM_LIMIT),
    )(e_h, n_s, n_r, u_h, xrs,
      dec1_we, dec1_ws, dec1_wr, dec1_wu, dec1_b, dec2_w, dec2_b)

    return out


# speedup vs baseline: 17.2750x; 1.6001x over previous
import functools

import jax
import jax.numpy as jnp
from jax import lax
from jax.experimental import pallas as pl
from jax.experimental.pallas import tpu as pltpu

_CompilerParams = getattr(pltpu, "CompilerParams", None) or getattr(pltpu, "TPUCompilerParams")

_VMEM_LIMIT = 64 * 1024 * 1024


def _edge_encode_kernel(ea_ref, xs_ref, xr_ref, u_ref,
                        we_ref, ws_ref, wr_ref, wu_ref, b_ref,
                        ehT_ref, ehTa_ref, esum_ref):
    c = jnp.dot(wu_ref[...], u_ref[...], preferred_element_type=jnp.float32) + b_ref[...]
    y = (jnp.dot(we_ref[...], ea_ref[...], preferred_element_type=jnp.float32)
         + jnp.dot(ws_ref[...], xs_ref[...], preferred_element_type=jnp.float32)
         + jnp.dot(wr_ref[...], xr_ref[...], preferred_element_type=jnp.float32)
         + c)
    ehT = jnp.maximum(y, 0.0)
    ehT_ref[...] = ehT
    te = ehT.shape[1]
    ehTa_ref[...] = jnp.concatenate(
        [ehT.astype(jnp.bfloat16),
         jnp.ones((1, te), jnp.bfloat16),
         jnp.zeros((7, te), jnp.bfloat16)], axis=0)

    @pl.when(pl.program_id(0) == 0)
    def _():
        esum_ref[...] = jnp.zeros_like(esum_ref)

    esum_ref[...] += jnp.sum(ehT, axis=1, keepdims=True)


def _node_kernel(x_ref, rcolb_ref, rrow_ref, srow_ref, ehTa_ref,
                 xrT_ref, xsT_ref, u_ref,
                 wagg_ref, wx_ref, wu_ref, b_ref,
                 nrT_ref, nsT_ref, corr_ref, nsum_ref):
    c = jnp.dot(wu_ref[...], u_ref[...], preferred_element_type=jnp.float32) + b_ref[...]

    baseN = jnp.maximum(
        lax.dot_general(wx_ref[...], x_ref[...], (((1,), (1,)), ((), ())),
                        preferred_element_type=jnp.float32) + c, 0.0)

    @pl.when(pl.program_id(0) == 0)
    def _():
        nsum_ref[...] = jnp.zeros_like(nsum_ref)
        corr_ref[...] = jnp.zeros_like(corr_ref)

    nsum_ref[...] += jnp.sum(baseN, axis=1, keepdims=True)

    rs = jnp.concatenate([rrow_ref[...], srow_ref[...]], axis=1)
    rcolb = rcolb_ref[...]
    nchunk = rs.shape[1] // 128
    mask = jnp.concatenate(
        [(rcolb == rs[:, k * 128:(k + 1) * 128]).astype(jnp.bfloat16)
         for k in range(nchunk)], axis=1)
    aggT2 = jnp.dot(ehTa_ref[...], mask, preferred_element_type=jnp.float32)

    te = rrow_ref.shape[1]
    aggrT = aggT2[:64, :te]
    multT = aggT2[64:65, :te]
    aggsT = aggT2[:64, te:]

    base_r = jnp.dot(wx_ref[...], xrT_ref[...], preferred_element_type=jnp.float32) + c
    base_s = jnp.dot(wx_ref[...], xsT_ref[...], preferred_element_type=jnp.float32) + c
    nrT = jnp.maximum(
        base_r + jnp.dot(wagg_ref[...], aggrT, preferred_element_type=jnp.float32), 0.0)
    nsT = jnp.maximum(
        base_s + jnp.dot(wagg_ref[...], aggsT, preferred_element_type=jnp.float32), 0.0)
    nrT_ref[...] = nrT
    nsT_ref[...] = nsT

    delta = (nrT - jnp.maximum(base_r, 0.0)) / multT
    corr_ref[...] += jnp.sum(delta, axis=1, keepdims=True)


def _global_kernel(esum_ref, nsum_ref, corr_ref, u_ref,
                   we_ref, wn_ref, wu_ref, b_ref, uh_ref, *, inv_e, inv_n):
    e_mean = esum_ref[...] * inv_e
    n_mean = (nsum_ref[...] + corr_ref[...]) * inv_n
    y = (jnp.dot(we_ref[...], e_mean, preferred_element_type=jnp.float32)
         + jnp.dot(wn_ref[...], n_mean, preferred_element_type=jnp.float32)
         + jnp.dot(wu_ref[...], u_ref[...], preferred_element_type=jnp.float32)
         + b_ref[...])
    uh_ref[...] = jnp.maximum(y, 0.0)


def _decode_kernel(ehT_ref, nsT_ref, nrT_ref, uh_ref, xrT_ref, xsT_ref,
                   w1e_ref, w1s_ref, w1r_ref, w1u_ref, b1_ref,
                   w2_ref, b2_ref, out_ref, *, out_dim):
    cu = jnp.dot(w1u_ref[...], uh_ref[...], preferred_element_type=jnp.float32) + b1_ref[...]
    h = (jnp.dot(w1e_ref[...], ehT_ref[...], preferred_element_type=jnp.float32)
         + jnp.dot(w1s_ref[...], nsT_ref[...], preferred_element_type=jnp.float32)
         + jnp.dot(w1r_ref[...], nrT_ref[...], preferred_element_type=jnp.float32)
         + cu)
    h = jnp.maximum(h, 0.0)
    o = jnp.dot(w2_ref[...], h, preferred_element_type=jnp.float32) + b2_ref[...]
    o1 = o[:out_dim, :]
    o2 = o[out_dim:, :]
    xr_row = xrT_ref[2:3, :]
    xs_row = xsT_ref[2:3, :]
    out_ref[...] = o1 * (xr_row - o2 * xs_row)


def _tile(dim, tile):
    t = min(tile, dim)
    if dim % t != 0 or t % 128 != 0:
        t = dim
    return t


def kernel(eb_we, eb_ws, eb_wr, eb_wu, eb_b,
           nb_wagg, nb_wx, nb_wu, nb_b,
           gb_we, gb_wn, gb_wu, gb_b,
           dec1_we, dec1_ws, dec1_wr, dec1_wu, dec1_b,
           dec2_w, dec2_b,
           x, edge_index, edge_attr, u):
    sind, rind = edge_index[0], edge_index[1]
    N, Fn = x.shape
    E, Fe = edge_attr.shape
    GH = u.shape[1]
    EH2 = eb_b.shape[1]
    NH2 = nb_b.shape[1]
    GH2 = gb_b.shape[1]
    OUT2 = dec2_b.shape[1]
    OUT = OUT2 // 2
    EH2A = EH2 + 8

    xs = x[sind]
    xr = x[rind]
    eaT = edge_attr.T
    xsT = xs.T
    xrT = xr.T
    uT = u.T
    rind32 = rind.astype(jnp.int32)
    sind32 = sind.astype(jnp.int32)
    rrow = rind32.reshape(1, E)
    srow = sind32.reshape(1, E)
    rcolb = jnp.tile(rind32.reshape(E, 1), (1, 128))

    tw = {
        "ewe": eb_we.T, "ews": eb_ws.T, "ewr": eb_wr.T, "ewu": eb_wu.T,
        "eb": eb_b.T,
        "nwagg": nb_wagg.T, "nwx": nb_wx.T, "nwu": nb_wu.T, "nb": nb_b.T,
        "gwe": gb_we.T, "gwn": gb_wn.T, "gwu": gb_wu.T, "gb": gb_b.T,
        "d1e": dec1_we.T, "d1s": dec1_ws.T, "d1r": dec1_wr.T,
        "d1u": dec1_wu.T, "d1b": dec1_b.T,
        "d2": dec2_w.T, "d2b": dec2_b.T,
    }

    te1 = _tile(E, 2048)
    g1 = E // te1
    ehT, ehTa, esumT = pl.pallas_call(
        _edge_encode_kernel,
        grid=(g1,),
        out_shape=(jax.ShapeDtypeStruct((EH2, E), jnp.float32),
                   jax.ShapeDtypeStruct((EH2A, E), jnp.bfloat16),
                   jax.ShapeDtypeStruct((EH2, 1), jnp.float32)),
        in_specs=[
            pl.BlockSpec((Fe, te1), lambda i: (0, i)),
            pl.BlockSpec((Fn, te1), lambda i: (0, i)),
            pl.BlockSpec((Fn, te1), lambda i: (0, i)),
            pl.BlockSpec((GH, 1), lambda i: (0, 0)),
            pl.BlockSpec((EH2, Fe), lambda i: (0, 0)),
            pl.BlockSpec((EH2, Fn), lambda i: (0, 0)),
            pl.BlockSpec((EH2, Fn), lambda i: (0, 0)),
            pl.BlockSpec((EH2, GH), lambda i: (0, 0)),
            pl.BlockSpec((EH2, 1), lambda i: (0, 0)),
        ],
        out_specs=(pl.BlockSpec((EH2, te1), lambda i: (0, i)),
                   pl.BlockSpec((EH2A, te1), lambda i: (0, i)),
                   pl.BlockSpec((EH2, 1), lambda i: (0, 0))),
        compiler_params=_CompilerParams(
            dimension_semantics=("arbitrary",),
            vmem_limit_bytes=_VMEM_LIMIT),
    )(eaT, xsT, xrT, uT, tw["ewe"], tw["ews"], tw["ewr"], tw["ewu"], tw["eb"])

    te2 = _tile(E, 256)
    g2 = E // te2
    tn = N // g2 if N % g2 == 0 else N
    if tn == N:
        g2 = 1
        te2 = E
    nrT, nsT, corrT, nsumT = pl.pallas_call(
        _node_kernel,
        grid=(g2,),
        out_shape=(jax.ShapeDtypeStruct((NH2, E), jnp.float32),
                   jax.ShapeDtypeStruct((NH2, E), jnp.float32),
                   jax.ShapeDtypeStruct((NH2, 1), jnp.float32),
                   jax.ShapeDtypeStruct((NH2, 1), jnp.float32)),
        in_specs=[
            pl.BlockSpec((tn, Fn), lambda i: (i, 0)),
            pl.BlockSpec((E, 128), lambda i: (0, 0)),
            pl.BlockSpec((1, te2), lambda i: (0, i)),
            pl.BlockSpec((1, te2), lambda i: (0, i)),
            pl.BlockSpec((EH2A, E), lambda i: (0, 0)),
            pl.BlockSpec((Fn, te2), lambda i: (0, i)),
            pl.BlockSpec((Fn, te2), lambda i: (0, i)),
            pl.BlockSpec((GH, 1), lambda i: (0, 0)),
            pl.BlockSpec((NH2, EH2), lambda i: (0, 0)),
            pl.BlockSpec((NH2, Fn), lambda i: (0, 0)),
            pl.BlockSpec((NH2, GH), lambda i: (0, 0)),
            pl.BlockSpec((NH2, 1), lambda i: (0, 0)),
        ],
        out_specs=(pl.BlockSpec((NH2, te2), lambda i: (0, i)),
                   pl.BlockSpec((NH2, te2), lambda i: (0, i)),
                   pl.BlockSpec((NH2, 1), lambda i: (0, 0)),
                   pl.BlockSpec((NH2, 1), lambda i: (0, 0))),
        compiler_params=_CompilerParams(
            dimension_semantics=("arbitrary",),
            vmem_limit_bytes=_VMEM_LIMIT),
    )(x, rcolb, rrow, srow, ehTa, xrT, xsT, uT,
      tw["nwagg"], tw["nwx"], tw["nwu"], tw["nb"])

    uhT = pl.pallas_call(
        functools.partial(_global_kernel, inv_e=1.0 / E, inv_n=1.0 / N),
        out_shape=jax.ShapeDtypeStruct((GH2, 1), jnp.float32),
        in_specs=[
            pl.BlockSpec((EH2, 1), lambda: (0, 0)),
            pl.BlockSpec((NH2, 1), lambda: (0, 0)),
            pl.BlockSpec((NH2, 1), lambda: (0, 0)),
            pl.BlockSpec((GH, 1), lambda: (0, 0)),
            pl.BlockSpec((GH2, EH2), lambda: (0, 0)),
            pl.BlockSpec((GH2, NH2), lambda: (0, 0)),
            pl.BlockSpec((GH2, GH), lambda: (0, 0)),
            pl.BlockSpec((GH2, 1), lambda: (0, 0)),
        ],
        out_specs=pl.BlockSpec((GH2, 1), lambda: (0, 0)),
    )(esumT, nsumT, corrT, uT, tw["gwe"], tw["gwn"], tw["gwu"], tw["gb"])

    te4 = _tile(E, 2048)
    outT = pl.pallas_call(
        functools.partial(_decode_kernel, out_dim=OUT),
        grid=(E // te4,),
        out_shape=jax.ShapeDtypeStruct((OUT, E), jnp.float32),
        in_specs=[
            pl.BlockSpec((EH2, te4), lambda i: (0, i)),
            pl.BlockSpec((NH2, te4), lambda i: (0, i)),
            pl.BlockSpec((NH2, te4), lambda i: (0, i)),
            pl.BlockSpec((GH2, 1), lambda i: (0, 0)),
            pl.BlockSpec((Fn, te4), lambda i: (0, i)),
            pl.BlockSpec((Fn, te4), lambda i: (0, i)),
            pl.BlockSpec((EH2, EH2), lambda i: (0, 0)),
            pl.BlockSpec((EH2, NH2), lambda i: (0, 0)),
            pl.BlockSpec((EH2, NH2), lambda i: (0, 0)),
            pl.BlockSpec((EH2, GH2), lambda i: (0, 0)),
            pl.BlockSpec((EH2, 1), lambda i: (0, 0)),
            pl.BlockSpec((OUT2, EH2), lambda i: (0, 0)),
            pl.BlockSpec((OUT2, 1), lambda i: (0, 0)),
        ],
        out_specs=pl.BlockSpec((OUT, te4), lambda i: (0, i)),
        compiler_params=_CompilerParams(
            dimension_semantics=("arbitrary",),
            vmem_limit_bytes=_VMEM_LIMIT),
    )(ehT, nsT, nrT, uhT, xrT, xsT,
      tw["d1e"], tw["d1s"], tw["d1r"], tw["d1u"], tw["d1b"],
      tw["d2"], tw["d2b"])

    return outT.T
